# baseline (speedup 1.0000x reference)
import functools

import jax
import jax.numpy as jnp
from jax import lax
from jax.experimental import pallas as pl
from jax.experimental.pallas import tpu as pltpu
from jax.experimental.pallas import tpu_sc as plsc

_B = 16384
_D = 64
_NC = 2
_NS = 16
_NW = _NC * _NS
_BPW = _B // _NW
_CH = 128
_NCHUNK = _BPW // _CH


def _neg_sqrt(s):
    sc = jnp.maximum(s, jnp.float32(1e-30))
    i = plsc.bitcast(sc, jnp.int32)
    i = jnp.int32(0x5F3759DF) - (i >> 1)
    y = plsc.bitcast(i, jnp.float32)
    half = sc * jnp.float32(0.5)
    for _ in range(3):
        y = y * (jnp.float32(1.5) - half * y * y)
    return -(s * y)


def _body(heads_hbm, rels_hbm, tails_hbm, ent_hbm, rel_hbm, out_hbm,
          hidx, ridx, tidx, hv, rv, tv, sums, outv, sem):
    wid = lax.axis_index("s") * _NC + lax.axis_index("c")
    base = wid * _BPW

    for j in range(_NCHUNK):
        off = base + j * _CH
        pltpu.sync_copy(heads_hbm.at[pl.ds(off, _CH)], hidx.at[j])
        pltpu.sync_copy(rels_hbm.at[pl.ds(off, _CH)], ridx.at[j])
        pltpu.sync_copy(tails_hbm.at[pl.ds(off, _CH)], tidx.at[j])

    copies = []
    for j in range(_NCHUNK):
        dst = pl.ds(j * _CH, _CH)
        copies.append(pltpu.async_copy(ent_hbm.at[hidx.at[j]], hv.at[dst], sem))
        copies.append(pltpu.async_copy(rel_hbm.at[ridx.at[j]], rv.at[dst], sem))
        copies.append(pltpu.async_copy(ent_hbm.at[tidx.at[j]], tv.at[dst], sem))
    for c in copies:
        c.wait()

    rows16 = lax.iota(jnp.int32, 16)

    def group(g, _):
        for k in range(16):
            i = g * 16 + k
            acc = jnp.zeros((16,), jnp.float32)
            for c in range(_D // 16):
                sl = pl.ds(c * 16, 16)
                d = hv[i, sl] + rv[i, sl] - tv[i, sl]
                acc = acc + d * d
            sums[k, pl.ds(0, 16)] = acc
        tot = jnp.zeros((16,), jnp.float32)
        for c in range(16):
            tot = tot + plsc.load_gather(
                sums, [rows16, jnp.full((16,), c, jnp.int32)])
        outv[pl.ds(g * 16, 16)] = _neg_sqrt(tot)
        return 0

    lax.fori_loop(0, _BPW // 16, group, 0)

    pltpu.sync_copy(outv, out_hbm.at[pl.ds(base, _BPW)])


@functools.partial(jax.jit)
def _transe_scores(heads, relations, tails, entity_emb, relation_emb):
    mesh = plsc.VectorSubcoreMesh(
        core_axis_name="c", subcore_axis_name="s",
        num_cores=_NC, num_subcores=_NS)
    run = pl.kernel(
        _body,
        out_type=jax.ShapeDtypeStruct((_B,), jnp.float32),
        mesh=mesh,
        compiler_params=pltpu.CompilerParams(
            needs_layout_passes=False, use_tc_tiling_on_sc=False),
        scratch_types=[
            pltpu.VMEM((_NCHUNK, _CH), jnp.int32),
            pltpu.VMEM((_NCHUNK, _CH), jnp.int32),
            pltpu.VMEM((_NCHUNK, _CH), jnp.int32),
            pltpu.VMEM((_BPW, _D), jnp.float32),
            pltpu.VMEM((_BPW, _D), jnp.float32),
            pltpu.VMEM((_BPW, _D), jnp.float32),
            pltpu.VMEM((16, 17), jnp.float32),
            pltpu.VMEM((_BPW,), jnp.float32),
            pltpu.SemaphoreType.DMA,
        ],
    )
    return run(heads, relations, tails, entity_emb, relation_emb)


def kernel(heads, relations, tails, entity_emb, relation_emb):
    return _transe_scores(heads.astype(jnp.int32), relations.astype(jnp.int32),
                          tails.astype(jnp.int32), entity_emb, relation_emb)
